# Initial kernel scaffold; baseline (speedup 1.0000x reference)
import jax
import jax.numpy as jnp
from jax.experimental import pallas as pl


def kernel(token_ids, table):
    raise NotImplementedError("write your pallas kernel here")


# speedup vs baseline: 1.8436x; 1.8436x over previous
import functools

import jax
import jax.numpy as jnp
from jax import lax
from jax.experimental import pallas as pl
from jax.experimental.pallas import tpu as pltpu
from jax.experimental.pallas import tpu_sc as plsc


def _make_gather(V, D, B):
    info = plsc.get_sparse_core_info()
    NC, NS = info.num_cores, info.num_subcores
    NW = NC * NS
    assert B % NW == 0
    b_per_w = B // NW
    C = 1024
    assert b_per_w % C == 0
    n_chunks = b_per_w // C
    mesh = plsc.VectorSubcoreMesh(core_axis_name="c", subcore_axis_name="s")

    @functools.partial(
        pl.kernel,
        mesh=mesh,
        out_type=jax.ShapeDtypeStruct((B, D), jnp.float32),
        compiler_params=pltpu.CompilerParams(use_tc_tiling_on_sc=False),
        scratch_types=[
            pltpu.VMEM((C,), jnp.int32),
            pltpu.VMEM((C, D), jnp.float32),
            pltpu.SemaphoreType.DMA,
        ],
    )
    def k(table_hbm, idx_hbm, out_hbm, idx_v, rows_v, sem):
        wid = lax.axis_index("s") * NC + lax.axis_index("c")
        base = wid * b_per_w

        def body(g, carry):
            off = base + g * C
            pltpu.sync_copy(idx_hbm.at[pl.ds(off, C)], idx_v)
            pltpu.async_copy(table_hbm.at[idx_v], rows_v, sem).wait()
            pltpu.sync_copy(rows_v, out_hbm.at[pl.ds(off, C)])
            return carry

        lax.fori_loop(0, n_chunks, body, 0)

    return k


def kernel(token_ids, table):
    batch, hist = token_ids.shape
    vocab, dim = table.shape
    ids = token_ids.reshape(-1).astype(jnp.int32)
    gather = _make_gather(vocab, dim, ids.shape[0])
    out = gather(table, ids)
    return out.reshape(batch, hist, dim)


# speedup vs baseline: 1.8602x; 1.0090x over previous
import functools

import jax
import jax.numpy as jnp
from jax import lax
from jax.experimental import pallas as pl
from jax.experimental.pallas import tpu as pltpu
from jax.experimental.pallas import tpu_sc as plsc

_CHUNK = 800
_NBUF = 2


def _make_gather(V, D, B):
    info = plsc.get_sparse_core_info()
    NC, NS = info.num_cores, info.num_subcores
    NW = NC * NS
    assert B % NW == 0
    b_per_w = B // NW
    C = _CHUNK
    assert b_per_w % (C * _NBUF) == 0
    n_chunks = b_per_w // C
    n_outer = n_chunks // _NBUF
    mesh = plsc.VectorSubcoreMesh(core_axis_name="c", subcore_axis_name="s")

    @functools.partial(
        pl.kernel,
        mesh=mesh,
        out_type=jax.ShapeDtypeStruct((B, D), jnp.float32),
        compiler_params=pltpu.CompilerParams(use_tc_tiling_on_sc=False),
        scratch_types=[
            pltpu.VMEM((n_chunks, C), jnp.int32),
            pltpu.VMEM((_NBUF, C, D), jnp.float32),
            pltpu.SemaphoreType.DMA,
            pltpu.SemaphoreType.DMA,
            pltpu.SemaphoreType.DMA,
            pltpu.SemaphoreType.DMA,
        ],
    )
    def k(table_hbm, idx_hbm, out_hbm, idx_v, rows_v, g0, g1, w0, w1):
        gs = (g0, g1)
        ws = (w0, w1)
        wid = lax.axis_index("s") * NC + lax.axis_index("c")
        base = wid * b_per_w

        pltpu.sync_copy(idx_hbm.at[pl.ds(wid * n_chunks, n_chunks)], idx_v)

        def gather(g, b):
            return pltpu.async_copy(table_hbm.at[idx_v.at[g]], rows_v.at[b], gs[b])

        def write(g, b):
            return pltpu.async_copy(
                rows_v.at[b], out_hbm.at[pl.ds(base + g * C, C)], ws[b]
            )

        for b in range(_NBUF):
            gather(b, b)

        def body(t, carry):
            for b in range(_NBUF):
                g = t * _NBUF + b
                pltpu.make_async_copy(
                    table_hbm.at[idx_v.at[g]], rows_v.at[b], gs[b]
                ).wait()
                write(g, b)

            @pl.when(t + 1 < n_outer)
            def _():
                for b in range(_NBUF):
                    g = t * _NBUF + b
                    pltpu.make_async_copy(
                        rows_v.at[b], out_hbm.at[pl.ds(base + g * C, C)], ws[b]
                    ).wait()
                    gather(g + _NBUF, b)

            return carry

        lax.fori_loop(0, n_outer, body, 0)

        for b in range(_NBUF):
            g = (n_outer - 1) * _NBUF + b
            pltpu.make_async_copy(
                rows_v.at[b], out_hbm.at[pl.ds(base + g * C, C)], ws[b]
            ).wait()

    return k


def kernel(token_ids, table):
    batch, hist = token_ids.shape
    vocab, dim = table.shape
    ids = token_ids.reshape(-1).astype(jnp.int32)
    b_total = ids.shape[0]
    ids2d = ids.reshape(b_total // _CHUNK, _CHUNK)
    gather = _make_gather(vocab, dim, b_total)
    out = gather(table, ids2d)
    return out.reshape(batch, hist, dim)


# speedup vs baseline: 2.1281x; 1.1440x over previous
import functools

import jax
import jax.numpy as jnp
from jax import lax
from jax.experimental import pallas as pl
from jax.experimental.pallas import tpu as pltpu
from jax.experimental.pallas import tpu_sc as plsc

_CHUNK = 800
_NBUF = 2


def _make_gather(V, D, B):
    info = plsc.get_sparse_core_info()
    NC, NS = info.num_cores, info.num_subcores
    NW = NC * NS
    assert B % NW == 0
    b_per_w = B // NW
    C = _CHUNK
    assert b_per_w % (C * _NBUF) == 0
    n_chunks = b_per_w // C
    n_outer = n_chunks // _NBUF
    mesh = plsc.VectorSubcoreMesh(core_axis_name="c", subcore_axis_name="s")

    @functools.partial(
        pl.kernel,
        mesh=mesh,
        out_type=jax.ShapeDtypeStruct((B, D), jnp.float32),
        compiler_params=pltpu.CompilerParams(use_tc_tiling_on_sc=False),
        scratch_types=[
            pltpu.VMEM((n_chunks, C), jnp.int32),
            pltpu.VMEM((_NBUF, C, D), jnp.float32),
            pltpu.SemaphoreType.DMA,
            pltpu.SemaphoreType.DMA,
            pltpu.SemaphoreType.DMA,
            pltpu.SemaphoreType.DMA,
        ],
    )
    def k(table_hbm, idx_hbm, out_hbm, idx_v, rows_v, g0, g1, w0, w1):
        gs = (g0, g1)
        ws = (w0, w1)
        wid = lax.axis_index("s") * NC + lax.axis_index("c")
        base = wid * b_per_w

        pltpu.sync_copy(idx_hbm.at[pl.ds(wid * n_chunks, n_chunks)], idx_v)

        def gather(g, b):
            return pltpu.async_copy(table_hbm.at[idx_v.at[g]], rows_v.at[b], gs[b])

        def write(g, b):
            return pltpu.async_copy(
                rows_v.at[b], out_hbm.at[pl.ds(base + g * C, C)], ws[b]
            )

        for b in range(_NBUF):
            gather(b, b)

        def body(t, carry):
            for b in range(_NBUF):
                g = t * _NBUF + b
                pltpu.make_async_copy(
                    table_hbm.at[idx_v.at[g]], rows_v.at[b], gs[b]
                ).wait()
                write(g, b)

            @pl.when(t + 1 < n_outer)
            def _():
                for b in range(_NBUF):
                    g = t * _NBUF + b
                    pltpu.make_async_copy(
                        rows_v.at[b], out_hbm.at[pl.ds(base + g * C, C)], ws[b]
                    ).wait()
                    gather(g + _NBUF, b)

            return carry

        lax.fori_loop(0, n_outer, body, 0)

        for b in range(_NBUF):
            g = (n_outer - 1) * _NBUF + b
            pltpu.make_async_copy(
                rows_v.at[b], out_hbm.at[pl.ds(base + g * C, C)], ws[b]
            ).wait()

    return k


def kernel(token_ids, table):
    batch, hist = token_ids.shape
    vocab, dim = table.shape
    ids = token_ids.reshape(-1).astype(jnp.int32)
    b_total = ids.shape[0]
    ids2d = ids.reshape(b_total // _CHUNK, _CHUNK)
    gather = _make_gather(vocab, dim, b_total)
    out = gather(table, ids2d)
    o1 = jax.lax.optimization_barrier(out.reshape(batch, hist * dim))
    return o1.reshape(batch, hist, dim)


# speedup vs baseline: 3.1368x; 1.4740x over previous
import functools

import jax
import jax.numpy as jnp
from jax import lax
from jax.experimental import pallas as pl
from jax.experimental.pallas import tpu as pltpu
from jax.experimental.pallas import tpu_sc as plsc

_CHUNK = 800
_NBUF = 2


def _make_gather(V, D, B):
    info = plsc.get_sparse_core_info()
    NC, NS = info.num_cores, info.num_subcores
    NW = NC * NS
    assert B % NW == 0
    b_per_w = B // NW
    C = _CHUNK
    assert b_per_w % (C * _NBUF) == 0
    n_chunks = b_per_w // C
    n_outer = n_chunks // _NBUF
    mesh = plsc.VectorSubcoreMesh(core_axis_name="c", subcore_axis_name="s")

    @functools.partial(
        pl.kernel,
        mesh=mesh,
        out_type=jax.ShapeDtypeStruct((B, D), jnp.float32),
        compiler_params=pltpu.CompilerParams(use_tc_tiling_on_sc=False),
        scratch_types=[
            pltpu.VMEM((n_chunks, C), jnp.int32),
            pltpu.VMEM((_NBUF, C, D), jnp.float32),
            pltpu.SemaphoreType.DMA,
            pltpu.SemaphoreType.DMA,
            pltpu.SemaphoreType.DMA,
            pltpu.SemaphoreType.DMA,
        ],
    )
    def k(table_hbm, idx_hbm, out_hbm, idx_v, rows_v, g0, g1, w0, w1):
        gs = (g0, g1)
        ws = (w0, w1)
        wid = lax.axis_index("s") * NC + lax.axis_index("c")
        base = wid * b_per_w

        pltpu.sync_copy(idx_hbm.at[pl.ds(wid * n_chunks, n_chunks)], idx_v)

        def gather(g, b):
            return pltpu.async_copy(table_hbm.at[idx_v.at[g]], rows_v.at[b], gs[b])

        def write(g, b):
            return pltpu.async_copy(
                rows_v.at[b], out_hbm.at[pl.ds(base + g * C, C)], ws[b]
            )

        for b in range(_NBUF):
            gather(b, b)

        def body(t, carry):
            for b in range(_NBUF):
                g = t * _NBUF + b
                pltpu.make_async_copy(
                    table_hbm.at[idx_v.at[g]], rows_v.at[b], gs[b]
                ).wait()
                write(g, b)

            @pl.when(t + 1 < n_outer)
            def _():
                for b in range(_NBUF):
                    g = t * _NBUF + b
                    pltpu.make_async_copy(
                        rows_v.at[b], out_hbm.at[pl.ds(base + g * C, C)], ws[b]
                    ).wait()
                    gather(g + _NBUF, b)

            return carry

        lax.fori_loop(0, n_outer, body, 0)

        for b in range(_NBUF):
            g = (n_outer - 1) * _NBUF + b
            pltpu.make_async_copy(
                rows_v.at[b], out_hbm.at[pl.ds(base + g * C, C)], ws[b]
            ).wait()

    return k


_BV2 = 8192


def _make_table_rowmajor(V, D):
    nb = (V + 2 * _BV2 - 1) // (2 * _BV2)
    last = V // _BV2

    def body(xa_ref, xb_ref, o_ref):
        o_ref[:, 0:D] = xa_ref[...].T
        o_ref[:, D : 2 * D] = xb_ref[...].T

    return pl.pallas_call(
        body,
        grid=(nb,),
        in_specs=[
            pl.BlockSpec((D, _BV2), lambda j: (0, jnp.minimum(2 * j, last))),
            pl.BlockSpec(
                (D, _BV2), lambda j: (0, jnp.minimum(2 * j + 1, last))
            ),
        ],
        out_specs=pl.BlockSpec((_BV2, 2 * D), lambda j: (j, 0)),
        out_shape=jax.ShapeDtypeStruct((nb * _BV2, 2 * D), jnp.float32),
    )


def kernel(token_ids, table):
    batch, hist = token_ids.shape
    vocab, dim = table.shape
    tt = _make_table_rowmajor(vocab, dim)(table.T, table.T)
    t_rm = tt.reshape(tt.shape[0] * 2, dim)
    ids0 = token_ids.astype(jnp.int32)
    sup = ids0 & ~(2 * _BV2 - 1)
    ids_p = sup + 2 * (ids0 & (_BV2 - 1)) + ((ids0 >> 13) & 1)
    ids = ids_p.reshape(-1)
    b_total = ids.shape[0]
    ids2d = ids.reshape(b_total // _CHUNK, _CHUNK)
    gather = _make_gather(vocab, dim, b_total)
    out = gather(t_rm, ids2d)
    o1 = jax.lax.optimization_barrier(out.reshape(batch, hist * dim))
    return o1.reshape(batch, hist, dim)
